# baseline (speedup 1.0000x reference)
import dataclasses

import jax
import jax.numpy as jnp
from jax import lax
from jax.experimental import pallas as pl
from jax.experimental.pallas import tpu as pltpu
from jax.experimental.pallas import tpu_sc as plsc

B = 1024
IN_F = 4096
OUT_F = 4096
K = 16
LANES = 16

BC = 16
NB = 512
NBP = NB + 1
N_CHUNKS = B // BC
NUM_WORKERS = 32
CHUNKS_PER_W = N_CHUNKS // NUM_WORKERS
NBLKS = OUT_F // NB
UNROLL = 2


def _splat(vec, lane):
    idx = jnp.full((LANES,), lane, jnp.int32)
    return vec.at[idx].get(mode="promise_in_bounds")


def _body(inpt_hbm, w_hbm, bias_hbm, mask_hbm, out_hbm,
          chunk_v, w_v, m_v, bias_v, out_v):
    c = lax.axis_index("c")
    s = lax.axis_index("s")
    wid = s * 2 + c

    pltpu.sync_copy(bias_hbm, bias_v)
    row_iota = lax.iota(jnp.int32, LANES)

    def chunk_body(ci, carry):
        b0 = (wid * CHUNKS_PER_W + ci) * BC
        pltpu.sync_copy(inpt_hbm.at[:, pl.ds(b0, BC)], chunk_v)

        def nb_body(nb, carry2):
            n0 = nb * NB
            pltpu.sync_copy(w_hbm.at[pl.ds(n0, NB), :], w_v)
            pltpu.sync_copy(mask_hbm.at[pl.ds(n0, NB), :], m_v)

            def n_body(ni, carry3):
                for j in range(UNROLL):
                    n = ni * UNROLL + j
                    g = n // LANES
                    m_row = m_v[n]
                    w_row = w_v[n]
                    bias_row = bias_v[pl.ds(n0 + (n // LANES) * LANES, LANES)]
                    accs = [jnp.zeros((LANES,), jnp.float32) for _ in range(4)]
                    for k in range(K):
                        m = m_row[k]
                        row = chunk_v[m]
                        accs[k % 4] = accs[k % 4] + row * _splat(w_row, k)
                    acc = ((accs[0] + accs[1]) + (accs[2] + accs[3])
                           + _splat(bias_row, n % LANES))
                    nvec = row_iota * 0 + n
                    plsc.store_scatter(out_v, [row_iota, nvec], acc)
                return carry3

            lax.fori_loop(0, NB // UNROLL, n_body, 0)
            pltpu.sync_copy(out_v.at[:, pl.ds(0, NB)],
                            out_hbm.at[pl.ds(b0, BC), pl.ds(n0, NB)])
            return carry2

        lax.fori_loop(0, NBLKS, nb_body, 0)
        return carry

    lax.fori_loop(0, CHUNKS_PER_W, chunk_body, 0)


@jax.jit
def kernel(input, condensed_weight, bias, input_mask):
    inpt = input.T
    maski = input_mask.astype(jnp.int32)
    mesh = plsc.VectorSubcoreMesh(core_axis_name="c", subcore_axis_name="s")
    cp = pltpu.CompilerParams()
    if "needs_layout_passes" in pltpu.CompilerParams.__dataclass_fields__:
        cp = dataclasses.replace(cp, needs_layout_passes=False)
    cp = dataclasses.replace(cp, use_tc_tiling_on_sc=False)
    f = pl.kernel(
        _body,
        out_type=jax.ShapeDtypeStruct((B, OUT_F), jnp.float32),
        mesh=mesh,
        scratch_types=[
            pltpu.VMEM((IN_F, BC), jnp.float32),
            pltpu.VMEM((NB, K), jnp.float32),
            pltpu.VMEM((NB, K), jnp.int32),
            pltpu.VMEM((OUT_F,), jnp.float32),
            pltpu.VMEM((BC, NBP), jnp.float32),
        ],
        compiler_params=cp,
    )
    return f(inpt, condensed_weight, bias, maski)


# speedup vs baseline: 1.3434x; 1.3434x over previous
import dataclasses

import jax
import jax.numpy as jnp
from jax import lax
from jax.experimental import pallas as pl
from jax.experimental.pallas import tpu as pltpu
from jax.experimental.pallas import tpu_sc as plsc

B = 1024
IN_F = 4096
OUT_F = 4096
K = 16
LANES = 16

NPAIR = 16
NB = 512
NUM_WORKERS = 32
GROUPS = NB // LANES
NBLKS = OUT_F // NB
PHALF = NPAIR // 2


def _body(pairs_hbm, wt_hbm, bias_hbm, maskt_hbm, out_hbm,
          chunk_v, w_v, m_v, bias_v, out_v):
    c = lax.axis_index("c")
    s = lax.axis_index("s")
    wid = s * 2 + c
    p0 = wid * NPAIR

    pltpu.sync_copy(bias_hbm, bias_v)
    pltpu.sync_copy(pairs_hbm.at[pl.ds(p0, NPAIR), :], chunk_v)

    def nb_body(nb, carry2):
        n0 = nb * NB
        pltpu.sync_copy(wt_hbm.at[:, pl.ds(n0, NB)], w_v)
        pltpu.sync_copy(maskt_hbm.at[:, pl.ds(n0, NB)], m_v)

        def g_body(g, carry3):
            gs = g * LANES
            bias_vec = bias_v[pl.ds(n0 + gs, LANES)]
            for ph in range(2):
                acc_lo = [bias_vec] * PHALF
                acc_hi = [bias_vec] * PHALF
                for k in range(K):
                    mk = m_v[k, pl.ds(gs, LANES)]
                    wk = w_v[k, pl.ds(gs, LANES)]
                    for pj in range(PHALF):
                        p = ph * PHALF + pj
                        pvec = jnp.full((LANES,), p, jnp.int32)
                        word = plsc.load_gather(chunk_v, [pvec, mk])
                        both = plsc.bitcast(word, jnp.bfloat16)
                        xlo, xhi = plsc.unpack(
                            both, format=plsc.PackFormat.INTERLEAVED)
                        acc_lo[pj] = acc_lo[pj] + xlo * wk
                        acc_hi[pj] = acc_hi[pj] + xhi * wk
                for pj in range(PHALF):
                    p = ph * PHALF + pj
                    out_v[2 * p, pl.ds(gs, LANES)] = acc_lo[pj]
                    out_v[2 * p + 1, pl.ds(gs, LANES)] = acc_hi[pj]
            return carry3

        lax.fori_loop(0, GROUPS, g_body, 0)
        pltpu.sync_copy(out_v, out_hbm.at[pl.ds(p0 * 2, NPAIR * 2),
                                          pl.ds(n0, NB)])
        return carry2

    lax.fori_loop(0, NBLKS, nb_body, 0)


@jax.jit
def kernel(input, condensed_weight, bias, input_mask):
    pairs = jax.lax.bitcast_convert_type(
        input.astype(jnp.bfloat16).reshape(B // 2, 2, IN_F).transpose(0, 2, 1),
        jnp.int32)
    wt = condensed_weight.T
    maskt = input_mask.T.astype(jnp.int32)
    mesh = plsc.VectorSubcoreMesh(core_axis_name="c", subcore_axis_name="s")
    cp = pltpu.CompilerParams()
    if "needs_layout_passes" in pltpu.CompilerParams.__dataclass_fields__:
        cp = dataclasses.replace(cp, needs_layout_passes=False)
    cp = dataclasses.replace(cp, use_tc_tiling_on_sc=False)
    f = pl.kernel(
        _body,
        out_type=jax.ShapeDtypeStruct((B, OUT_F), jnp.float32),
        mesh=mesh,
        scratch_types=[
            pltpu.VMEM((NPAIR, IN_F), jnp.int32),
            pltpu.VMEM((K, NB), jnp.float32),
            pltpu.VMEM((K, NB), jnp.int32),
            pltpu.VMEM((OUT_F,), jnp.float32),
            pltpu.VMEM((2 * NPAIR, NB), jnp.float32),
        ],
        compiler_params=cp,
    )
    return f(pairs, wt, bias, maskt)
